# baseline (speedup 1.0000x reference)
import jax
import jax.numpy as jnp
from jax.experimental import pallas as pl


_S_BLK = 512


def _pe_add_kernel(x_ref, w_ref, o_ref):
    w = w_ref[...]
    d = w.shape[1]
    for b in range(x_ref.shape[1] // d):
        sl = pl.ds(b * d, d)
        o_ref[:, sl] = x_ref[:, sl] + w


def kernel(x, pos_embed_weight):
    seq_len, batch, d_model = x.shape
    x2 = x.reshape(seq_len, batch * d_model)
    grid = (seq_len // _S_BLK,)
    out2 = pl.pallas_call(
        _pe_add_kernel,
        grid=grid,
        in_specs=[
            pl.BlockSpec((_S_BLK, batch * d_model), lambda i: (i, 0)),
            pl.BlockSpec((_S_BLK, d_model), lambda i: (i, 0)),
        ],
        out_specs=pl.BlockSpec((_S_BLK, batch * d_model), lambda i: (i, 0)),
        out_shape=jax.ShapeDtypeStruct((seq_len, batch * d_model), x.dtype),
    )(x2, pos_embed_weight)
    return out2.reshape(seq_len, batch, d_model)
